# baseline (device time: 12984 ns/iter reference)
import jax
import jax.numpy as jnp
from jax import lax
from jax.experimental import pallas as pl
from jax.experimental.pallas import tpu as pltpu

T = 256
D = 512
V_LOCAL = 4096


def kernel(x, W, labels):

    def body(x_ref, w_ref, lab_ref, out_ref, payload, recv, send_sems, recv_sems):
        my_x = lax.axis_index("x")
        my_y = lax.axis_index("y")
        my_z = lax.axis_index("z")

        barrier = pltpu.get_barrier_semaphore()
        for k in (1, 2, 3):
            pl.semaphore_signal(
                barrier,
                inc=1,
                device_id=(my_x, my_y, my_z ^ k),
                device_id_type=pl.DeviceIdType.MESH,
            )

        logits = jnp.dot(
            x_ref[...].astype(jnp.bfloat16),
            w_ref[...].astype(jnp.bfloat16),
            preferred_element_type=jnp.float32,
        ).astype(jnp.bfloat16)
        e = jnp.exp(logits)
        col = lax.broadcasted_iota(jnp.int16, (T, V_LOCAL), 1)
        lab2 = jnp.reshape(lab_ref[...], (T, 1))
        sel = col == (lab2 - my_z * V_LOCAL).astype(jnp.int16)
        masked = jnp.where(sel, e, jnp.bfloat16(0.0))
        s_col = jnp.sum(e, axis=1, keepdims=True, dtype=jnp.float32)
        q_col = jnp.sum(masked, axis=1, keepdims=True, dtype=jnp.float32)
        payload[...] = jnp.concatenate([s_col, q_col], axis=1).T

        pl.semaphore_wait(barrier, 3)

        rdmas = []
        for k in (3, 2, 1):
            rdma = pltpu.make_async_remote_copy(
                src_ref=payload,
                dst_ref=recv.at[k - 1],
                send_sem=send_sems.at[k - 1],
                recv_sem=recv_sems.at[k - 1],
                device_id=(my_x, my_y, my_z ^ k),
                device_id_type=pl.DeviceIdType.MESH,
            )
            rdma.start()
            rdmas.append(rdma)
        for rdma in rdmas:
            rdma.wait_recv()
        nll = (
            jnp.log(payload[0:1, :] + recv[0, 0:1, :] + recv[1, 0:1, :] + recv[2, 0:1, :])
            - jnp.log(payload[1:2, :] + recv[0, 1:2, :] + recv[1, 1:2, :] + recv[2, 1:2, :])
        )
        out_ref[...] = jnp.reshape(nll, (T,))
        for rdma in rdmas:
            rdma.wait_send()

    out = pl.pallas_call(
        body,
        out_shape=jax.ShapeDtypeStruct((T,), jnp.float32),
        in_specs=[pl.BlockSpec(memory_space=pltpu.VMEM)] * 3,
        out_specs=pl.BlockSpec(memory_space=pltpu.VMEM),
        scratch_shapes=[
            pltpu.VMEM((2, T), jnp.float32),
            pltpu.VMEM((3, 2, T), jnp.float32),
            pltpu.SemaphoreType.DMA((3,)),
            pltpu.SemaphoreType.DMA((3,)),
        ],
        compiler_params=pltpu.CompilerParams(collective_id=0),
    )(x, W, labels)
    return out
